# baseline (speedup 1.0000x reference)
import jax
import jax.numpy as jnp
from jax.experimental import pallas as pl
from jax.experimental.pallas import tpu as pltpu

_T = 0.07
_NBUF = 16


_CHUNK = 16


def _body(labr_ref, labc_ref, q_ref, k_ref, queue_ref, out_ref,
          bufs_ref, lneg_ref, lpp_ref, con_ref, pvm_ref, psm_ref,
          sem_ref, psem_ref):
    n, d = q_ref.shape
    i32 = jnp.int32

    labr = labr_ref[...]
    labc = labc_ref[...]
    ior = jax.lax.broadcasted_iota(i32, (n, n), 0)
    ioc = jax.lax.broadcasted_iota(i32, (n, n), 1)
    eqm = (labc == labr)
    ltm = (labc > labr) | (eqm & (ior > ioc))
    pos_c = jnp.sum(ltm.astype(i32), axis=1, keepdims=True)
    pos_r = jnp.sum(ltm.astype(i32), axis=0, keepdims=True)
    pos_r = (n - 1) - pos_r
    before = eqm & (ior > ioc)
    rf_c = (jnp.sum(before.astype(i32), axis=1, keepdims=True) == 0)
    after = eqm & (ior < ioc)
    rf_r = (jnp.sum(after.astype(i32), axis=0, keepdims=True) == 0)
    em = (pos_c == ioc)
    order_r = jnp.sum(jnp.where(em, ior, 0), axis=0, keepdims=True)
    first_r = jnp.sum((em & rf_c).astype(i32), axis=0, keepdims=True)
    segid_r = jnp.sum((rf_c & (pos_c <= ioc)).astype(i32),
                      axis=0, keepdims=True) - 1
    nseg = jnp.sum(rf_c.astype(i32))
    nseg_r = jnp.zeros((1, n), i32) + nseg
    segrow_c = jnp.sum((rf_r & (pos_r <= pos_c)).astype(i32),
                       axis=1, keepdims=True) - 1
    segcls_r = jnp.sum(jnp.where(rf_c & (segrow_c == ioc), labc, 0),
                       axis=0, keepdims=True)

    pvm_ref[pl.ds(0, 1), :] = segcls_r
    pvm_ref[pl.ds(1, 1), :] = segid_r
    pvm_ref[pl.ds(2, 1), :] = first_r
    pvm_ref[pl.ds(3, 1), :] = nseg_r
    pvm_ref[pl.ds(4, 1), :] = order_r
    copy = pltpu.make_async_copy(pvm_ref, psm_ref, psem_ref)
    copy.start()
    copy.wait()

    nsegv = psm_ref[3, 0]

    def _issue(s):
        c = psm_ref[0, s]
        pltpu.make_async_copy(
            queue_ref.at[c], bufs_ref.at[s % _NBUF], sem_ref.at[s % _NBUF]
        ).start()

    _issue(0)
    for j in range(1, _NBUF - 1):
        @pl.when(nsegv > j)
        def _ij(j=j):
            _issue(j)

    def _step(i, carry):
        s = psm_ref[1, i]
        r = psm_ref[4, i]

        @pl.when(psm_ref[2, i] == 1)
        def _seg_start():
            pltpu.make_async_copy(
                queue_ref.at[psm_ref[0, s]],
                bufs_ref.at[s % _NBUF],
                sem_ref.at[s % _NBUF],
            ).wait()

            @pl.when(s + _NBUF - 1 < nsegv)
            def _prefetch():
                _issue(s + _NBUF - 1)

        qrow = q_ref[pl.ds(r, 1), :] * (1.0 / _T)
        krow = k_ref[pl.ds(r, 1), :]
        res = jax.lax.dot_general(
            qrow, bufs_ref[s % _NBUF], (((1,), (0,)), ((), ())),
            preferred_element_type=jnp.float32,
        )
        lneg_ref[pl.ds(i, 1), :] = res
        lpp_ref[pl.ds(i, 1), :] = qrow * krow

        @pl.when(jax.lax.rem(i, _CHUNK) == _CHUNK - 1)
        def _chunk():
            base = (i // _CHUNK) * _CHUNK
            ln = lneg_ref[pl.ds(base, _CHUNK), :]
            lp = jnp.sum(lpp_ref[pl.ds(base, _CHUNK), :],
                         axis=1, keepdims=True)
            m = jnp.maximum(jnp.max(ln, axis=1, keepdims=True), lp)
            se = (jnp.sum(jnp.exp(ln - m), axis=1, keepdims=True)
                  + jnp.exp(lp - m))
            con_ref[pl.ds(base, _CHUNK), :] = m + jnp.log(se) - lp

        return carry

    jax.lax.fori_loop(0, n, _step, 0)

    loss = jnp.sum(con_ref[...]) * (1.0 / n)
    out_ref[...] = jnp.full((1, 1), loss, dtype=jnp.float32)


def kernel(q, k, weight, cls_labels, queue):
    del weight
    n, d = q.shape
    c, _, kq = queue.shape

    labels = cls_labels.astype(jnp.int32)
    labr = labels.reshape(1, n)
    labc = labels.reshape(n, 1)

    out = pl.pallas_call(
        _body,
        grid=(1,),
        in_specs=[
            pl.BlockSpec((1, n), lambda i: (0, 0)),
            pl.BlockSpec((n, 1), lambda i: (0, 0)),
            pl.BlockSpec((n, d), lambda i: (0, 0)),
            pl.BlockSpec((n, d), lambda i: (0, 0)),
            pl.BlockSpec(memory_space=pl.ANY),
        ],
        out_specs=pl.BlockSpec((1, 1), lambda i: (0, 0)),
        scratch_shapes=[
            pltpu.VMEM((_NBUF, d, kq), jnp.float32),
            pltpu.VMEM((n, kq), jnp.float32),
            pltpu.VMEM((n, d), jnp.float32),
            pltpu.VMEM((n, 1), jnp.float32),
            pltpu.VMEM((8, n), jnp.int32),
            pltpu.SMEM((8, n), jnp.int32),
            pltpu.SemaphoreType.DMA((_NBUF,)),
            pltpu.SemaphoreType.DMA,
        ],
        out_shape=jax.ShapeDtypeStruct((1, 1), jnp.float32),
    )(labr, labc, q, k, queue)
    return out[0, 0]


# speedup vs baseline: 1.0446x; 1.0446x over previous
import jax
import jax.numpy as jnp
from jax.experimental import pallas as pl
from jax.experimental.pallas import tpu as pltpu

_T = 0.07
_NBUF = 8


_CHUNK = 64


def _body(labr_ref, labc_ref, q_ref, k_ref, queue_ref, out_ref,
          bufs_ref, lneg_ref, lpp_ref, con_ref, pvm_ref, psm_ref,
          sem_ref, psem_ref):
    n, d = q_ref.shape
    i32 = jnp.int32

    labr = labr_ref[...]
    labc = labc_ref[...]
    ior = jax.lax.broadcasted_iota(i32, (n, n), 0)
    ioc = jax.lax.broadcasted_iota(i32, (n, n), 1)
    eqm = (labc == labr)
    ltm = (labc > labr) | (eqm & (ior > ioc))
    pos_c = jnp.sum(ltm.astype(i32), axis=1, keepdims=True)
    pos_r = jnp.sum(ltm.astype(i32), axis=0, keepdims=True)
    pos_r = (n - 1) - pos_r
    before = eqm & (ior > ioc)
    rf_c = (jnp.sum(before.astype(i32), axis=1, keepdims=True) == 0)
    after = eqm & (ior < ioc)
    rf_r = (jnp.sum(after.astype(i32), axis=0, keepdims=True) == 0)
    em = (pos_c == ioc)
    order_r = jnp.sum(jnp.where(em, ior, 0), axis=0, keepdims=True)
    first_r = jnp.sum((em & rf_c).astype(i32), axis=0, keepdims=True)
    segid_r = jnp.sum((rf_c & (pos_c <= ioc)).astype(i32),
                      axis=0, keepdims=True) - 1
    nseg = jnp.sum(rf_c.astype(i32))
    nseg_r = jnp.zeros((1, n), i32) + nseg
    segrow_c = jnp.sum((rf_r & (pos_r <= pos_c)).astype(i32),
                       axis=1, keepdims=True) - 1
    segcls_r = jnp.sum(jnp.where(rf_c & (segrow_c == ioc), labc, 0),
                       axis=0, keepdims=True)

    pvm_ref[pl.ds(0, 1), :] = segcls_r
    pvm_ref[pl.ds(1, 1), :] = segid_r
    pvm_ref[pl.ds(2, 1), :] = first_r
    pvm_ref[pl.ds(3, 1), :] = nseg_r
    pvm_ref[pl.ds(4, 1), :] = order_r
    copy = pltpu.make_async_copy(pvm_ref, psm_ref, psem_ref)
    copy.start()
    copy.wait()

    nsegv = psm_ref[3, 0]

    def _issue(s):
        c = psm_ref[0, s]
        pltpu.make_async_copy(
            queue_ref.at[c], bufs_ref.at[s % _NBUF], sem_ref.at[s % _NBUF]
        ).start()

    _issue(0)
    for j in range(1, _NBUF - 1):
        @pl.when(nsegv > j)
        def _ij(j=j):
            _issue(j)

    def _step(i, carry):
        s = psm_ref[1, i]
        r = psm_ref[4, i]

        @pl.when(psm_ref[2, i] == 1)
        def _seg_start():
            pltpu.make_async_copy(
                queue_ref.at[psm_ref[0, s]],
                bufs_ref.at[s % _NBUF],
                sem_ref.at[s % _NBUF],
            ).wait()

            @pl.when(s + _NBUF - 1 < nsegv)
            def _prefetch():
                _issue(s + _NBUF - 1)

        qrow = q_ref[pl.ds(r, 1), :] * (1.0 / _T)
        krow = k_ref[pl.ds(r, 1), :]
        res = jax.lax.dot_general(
            qrow, bufs_ref[s % _NBUF], (((1,), (0,)), ((), ())),
            preferred_element_type=jnp.float32,
        )
        lneg_ref[pl.ds(i, 1), :] = res
        lpp_ref[pl.ds(i, 1), :] = qrow * krow

        @pl.when(jax.lax.rem(i, _CHUNK) == _CHUNK - 1)
        def _chunk():
            base = (i // _CHUNK) * _CHUNK
            ln = lneg_ref[pl.ds(base, _CHUNK), :]
            lp = jnp.sum(lpp_ref[pl.ds(base, _CHUNK), :],
                         axis=1, keepdims=True)
            m = jnp.maximum(jnp.max(ln, axis=1, keepdims=True), lp)
            se = (jnp.sum(jnp.exp(ln - m), axis=1, keepdims=True)
                  + jnp.exp(lp - m))
            con_ref[pl.ds(base, _CHUNK), :] = m + jnp.log(se) - lp

        return carry

    jax.lax.fori_loop(0, n, _step, 0)

    loss = jnp.sum(con_ref[...]) * (1.0 / n)
    out_ref[...] = jnp.full((1, 1), loss, dtype=jnp.float32)


def kernel(q, k, weight, cls_labels, queue):
    del weight
    n, d = q.shape
    c, _, kq = queue.shape

    labels = cls_labels.astype(jnp.int32)
    labr = labels.reshape(1, n)
    labc = labels.reshape(n, 1)

    out = pl.pallas_call(
        _body,
        grid=(1,),
        in_specs=[
            pl.BlockSpec((1, n), lambda i: (0, 0)),
            pl.BlockSpec((n, 1), lambda i: (0, 0)),
            pl.BlockSpec((n, d), lambda i: (0, 0)),
            pl.BlockSpec((n, d), lambda i: (0, 0)),
            pl.BlockSpec(memory_space=pl.ANY),
        ],
        out_specs=pl.BlockSpec((1, 1), lambda i: (0, 0)),
        scratch_shapes=[
            pltpu.VMEM((_NBUF, d, kq), jnp.float32),
            pltpu.VMEM((n, kq), jnp.float32),
            pltpu.VMEM((n, d), jnp.float32),
            pltpu.VMEM((n, 1), jnp.float32),
            pltpu.VMEM((8, n), jnp.int32),
            pltpu.SMEM((8, n), jnp.int32),
            pltpu.SemaphoreType.DMA((_NBUF,)),
            pltpu.SemaphoreType.DMA,
        ],
        out_shape=jax.ShapeDtypeStruct((1, 1), jnp.float32),
    )(labr, labc, q, k, queue)
    return out[0, 0]


# speedup vs baseline: 1.0451x; 1.0005x over previous
import jax
import jax.numpy as jnp
from jax.experimental import pallas as pl
from jax.experimental.pallas import tpu as pltpu

_T = 0.07
_NBUF = 8


_CHUNK = 32


def _body(labsp_ref, labr_ref, labc_ref, q_ref, k_ref, queue_ref, out_ref,
          bufs_ref, lneg_ref, lpp_ref, con_ref, pvm_ref, psm_ref,
          sem_ref, psem_ref):
    n, d = q_ref.shape
    i32 = jnp.int32

    pltpu.make_async_copy(
        queue_ref.at[labsp_ref[0]], bufs_ref.at[0], sem_ref.at[0]
    ).start()

    labr = labr_ref[...]
    labc = labc_ref[...]
    lab0 = labr_ref[0:1, 0:1]
    keyr = jnp.where(labr == lab0, -1, labr)
    keyc = jnp.where(labc == lab0, -1, labc)
    ior = jax.lax.broadcasted_iota(i32, (n, n), 0)
    ioc = jax.lax.broadcasted_iota(i32, (n, n), 1)
    eqm = (keyc == keyr)
    ltm = (keyc > keyr) | (eqm & (ior > ioc))
    pos_c = jnp.sum(ltm.astype(i32), axis=1, keepdims=True)
    pos_r = jnp.sum(ltm.astype(i32), axis=0, keepdims=True)
    pos_r = (n - 1) - pos_r
    before = eqm & (ior > ioc)
    rf_c = (jnp.sum(before.astype(i32), axis=1, keepdims=True) == 0)
    after = eqm & (ior < ioc)
    rf_r = (jnp.sum(after.astype(i32), axis=0, keepdims=True) == 0)
    em = (pos_c == ioc)
    order_r = jnp.sum(jnp.where(em, ior, 0), axis=0, keepdims=True)
    first_r = jnp.sum((em & rf_c).astype(i32), axis=0, keepdims=True)
    segid_r = jnp.sum((rf_c & (pos_c <= ioc)).astype(i32),
                      axis=0, keepdims=True) - 1
    nseg = jnp.sum(rf_c.astype(i32))
    nseg_r = jnp.zeros((1, n), i32) + nseg
    segrow_c = jnp.sum((rf_r & (pos_r <= pos_c)).astype(i32),
                       axis=1, keepdims=True) - 1
    segcls_r = jnp.sum(jnp.where(rf_c & (segrow_c == ioc), labc, 0),
                       axis=0, keepdims=True)

    pvm_ref[pl.ds(0, 1), :] = segcls_r
    pvm_ref[pl.ds(1, 1), :] = segid_r
    pvm_ref[pl.ds(2, 1), :] = first_r
    pvm_ref[pl.ds(3, 1), :] = nseg_r
    pvm_ref[pl.ds(4, 1), :] = order_r
    copy = pltpu.make_async_copy(pvm_ref, psm_ref, psem_ref)
    copy.start()
    copy.wait()

    nsegv = psm_ref[3, 0]

    def _issue(s):
        c = psm_ref[0, s]
        pltpu.make_async_copy(
            queue_ref.at[c], bufs_ref.at[s % _NBUF], sem_ref.at[s % _NBUF]
        ).start()

    for j in range(1, _NBUF - 1):
        @pl.when(nsegv > j)
        def _ij(j=j):
            _issue(j)

    def _step(i, carry):
        s = psm_ref[1, i]
        r = psm_ref[4, i]

        @pl.when(psm_ref[2, i] == 1)
        def _seg_start():
            pltpu.make_async_copy(
                queue_ref.at[psm_ref[0, s]],
                bufs_ref.at[s % _NBUF],
                sem_ref.at[s % _NBUF],
            ).wait()

            @pl.when(s + _NBUF - 1 < nsegv)
            def _prefetch():
                _issue(s + _NBUF - 1)

        qrow = q_ref[pl.ds(r, 1), :] * (1.0 / _T)
        krow = k_ref[pl.ds(r, 1), :]
        res = jax.lax.dot_general(
            qrow, bufs_ref[s % _NBUF], (((1,), (0,)), ((), ())),
            preferred_element_type=jnp.float32,
        )
        lneg_ref[pl.ds(i, 1), :] = res
        lpp_ref[pl.ds(i, 1), :] = qrow * krow

        @pl.when(jax.lax.rem(i, _CHUNK) == _CHUNK - 1)
        def _chunk():
            base = (i // _CHUNK) * _CHUNK
            ln = lneg_ref[pl.ds(base, _CHUNK), :]
            lp = jnp.sum(lpp_ref[pl.ds(base, _CHUNK), :],
                         axis=1, keepdims=True)
            m = jnp.maximum(jnp.max(ln, axis=1, keepdims=True), lp)
            se = (jnp.sum(jnp.exp(ln - m), axis=1, keepdims=True)
                  + jnp.exp(lp - m))
            con_ref[pl.ds(base, _CHUNK), :] = m + jnp.log(se) - lp

        return carry

    jax.lax.fori_loop(0, n, _step, 0)

    loss = jnp.sum(con_ref[...]) * (1.0 / n)
    out_ref[...] = jnp.full((1, 1), loss, dtype=jnp.float32)


def kernel(q, k, weight, cls_labels, queue):
    del weight
    n, d = q.shape
    c, _, kq = queue.shape

    labels = cls_labels.astype(jnp.int32)
    labr = labels.reshape(1, n)
    labc = labels.reshape(n, 1)

    grid_spec = pltpu.PrefetchScalarGridSpec(
        num_scalar_prefetch=1,
        grid=(1,),
        in_specs=[
            pl.BlockSpec((1, n), lambda i, lab: (0, 0)),
            pl.BlockSpec((n, 1), lambda i, lab: (0, 0)),
            pl.BlockSpec((n, d), lambda i, lab: (0, 0)),
            pl.BlockSpec((n, d), lambda i, lab: (0, 0)),
            pl.BlockSpec(memory_space=pl.ANY),
        ],
        out_specs=pl.BlockSpec((1, 1), lambda i, lab: (0, 0)),
        scratch_shapes=[
            pltpu.VMEM((_NBUF, d, kq), jnp.float32),
            pltpu.VMEM((n, kq), jnp.float32),
            pltpu.VMEM((n, d), jnp.float32),
            pltpu.VMEM((n, 1), jnp.float32),
            pltpu.VMEM((8, n), jnp.int32),
            pltpu.SMEM((8, n), jnp.int32),
            pltpu.SemaphoreType.DMA((_NBUF,)),
            pltpu.SemaphoreType.DMA,
        ],
    )
    out = pl.pallas_call(
        _body,
        grid_spec=grid_spec,
        out_shape=jax.ShapeDtypeStruct((1, 1), jnp.float32),
    )(labels, labr, labc, q, k, queue)
    return out[0, 0]


# speedup vs baseline: 1.0465x; 1.0014x over previous
import jax
import jax.numpy as jnp
from jax.experimental import pallas as pl
from jax.experimental.pallas import tpu as pltpu

_T = 0.07
_NBUF = 8


_CHUNK = 32


def _body(labr_ref, labc_ref, q_ref, k_ref, queue_ref, out_ref,
          bufs_ref, lneg_ref, lpp_ref, con_ref, pvm_ref, psm_ref,
          sem_ref, psem_ref):
    n, d = q_ref.shape
    i32 = jnp.int32

    labr = labr_ref[...]
    labc = labc_ref[...]
    ior = jax.lax.broadcasted_iota(i32, (n, n), 0)
    ioc = jax.lax.broadcasted_iota(i32, (n, n), 1)
    eqm = (labc == labr)
    ltm = (labc > labr) | (eqm & (ior > ioc))
    pos_c = jnp.sum(ltm.astype(i32), axis=1, keepdims=True)
    pos_r = jnp.sum(ltm.astype(i32), axis=0, keepdims=True)
    pos_r = (n - 1) - pos_r
    before = eqm & (ior > ioc)
    rf_c = (jnp.sum(before.astype(i32), axis=1, keepdims=True) == 0)
    after = eqm & (ior < ioc)
    rf_r = (jnp.sum(after.astype(i32), axis=0, keepdims=True) == 0)
    em = (pos_c == ioc)
    order_r = jnp.sum(jnp.where(em, ior, 0), axis=0, keepdims=True)
    first_r = jnp.sum((em & rf_c).astype(i32), axis=0, keepdims=True)
    segid_r = jnp.sum((rf_c & (pos_c <= ioc)).astype(i32),
                      axis=0, keepdims=True) - 1
    nseg = jnp.sum(rf_c.astype(i32))
    nseg_r = jnp.zeros((1, n), i32) + nseg
    segrow_c = jnp.sum((rf_r & (pos_r <= pos_c)).astype(i32),
                       axis=1, keepdims=True) - 1
    segcls_r = jnp.sum(jnp.where(rf_c & (segrow_c == ioc), labc, 0),
                       axis=0, keepdims=True)

    pvm_ref[pl.ds(0, 1), :] = segcls_r
    pvm_ref[pl.ds(1, 1), :] = segid_r
    pvm_ref[pl.ds(2, 1), :] = first_r
    pvm_ref[pl.ds(3, 1), :] = nseg_r
    pvm_ref[pl.ds(4, 1), :] = order_r
    copy = pltpu.make_async_copy(pvm_ref, psm_ref, psem_ref)
    copy.start()
    copy.wait()

    nsegv = psm_ref[3, 0]

    def _issue(s):
        c = psm_ref[0, s]
        pltpu.make_async_copy(
            queue_ref.at[c], bufs_ref.at[s % _NBUF], sem_ref.at[s % _NBUF]
        ).start()

    _issue(0)
    for j in range(1, _NBUF - 1):
        @pl.when(nsegv > j)
        def _ij(j=j):
            _issue(j)

    def _step(i, carry):
        s = psm_ref[1, i]
        r = psm_ref[4, i]

        @pl.when(psm_ref[2, i] == 1)
        def _seg_start():
            pltpu.make_async_copy(
                queue_ref.at[psm_ref[0, s]],
                bufs_ref.at[s % _NBUF],
                sem_ref.at[s % _NBUF],
            ).wait()

            @pl.when(s + _NBUF - 1 < nsegv)
            def _prefetch():
                _issue(s + _NBUF - 1)

        qrow = q_ref[pl.ds(r, 1), :] * (1.0 / _T)
        krow = k_ref[pl.ds(r, 1), :]
        res = jax.lax.dot_general(
            qrow, bufs_ref[s % _NBUF], (((1,), (0,)), ((), ())),
            preferred_element_type=jnp.float32,
        )
        lneg_ref[pl.ds(i, 1), :] = res
        lpp_ref[pl.ds(i, 1), :] = qrow * krow

        @pl.when(jax.lax.rem(i, _CHUNK) == _CHUNK - 1)
        def _chunk():
            base = (i // _CHUNK) * _CHUNK
            ln = lneg_ref[pl.ds(base, _CHUNK), :]
            lp = jnp.sum(lpp_ref[pl.ds(base, _CHUNK), :],
                         axis=1, keepdims=True)
            m = jnp.maximum(jnp.max(ln, axis=1, keepdims=True), lp)
            se = (jnp.sum(jnp.exp(ln - m), axis=1, keepdims=True)
                  + jnp.exp(lp - m))
            con_ref[pl.ds(base, _CHUNK), :] = m + jnp.log(se) - lp

        return carry

    jax.lax.fori_loop(0, n, _step, 0)

    loss = jnp.sum(con_ref[...]) * (1.0 / n)
    out_ref[...] = jnp.full((1, 1), loss, dtype=jnp.float32)


def kernel(q, k, weight, cls_labels, queue):
    del weight
    n, d = q.shape
    c, _, kq = queue.shape

    labels = cls_labels.astype(jnp.int32)
    labr = labels.reshape(1, n)
    labc = labels.reshape(n, 1)

    out = pl.pallas_call(
        _body,
        grid=(1,),
        in_specs=[
            pl.BlockSpec((1, n), lambda i: (0, 0)),
            pl.BlockSpec((n, 1), lambda i: (0, 0)),
            pl.BlockSpec((n, d), lambda i: (0, 0)),
            pl.BlockSpec((n, d), lambda i: (0, 0)),
            pl.BlockSpec(memory_space=pl.ANY),
        ],
        out_specs=pl.BlockSpec((1, 1), lambda i: (0, 0)),
        scratch_shapes=[
            pltpu.VMEM((_NBUF, d, kq), jnp.float32),
            pltpu.VMEM((n, kq), jnp.float32),
            pltpu.VMEM((n, d), jnp.float32),
            pltpu.VMEM((n, 1), jnp.float32),
            pltpu.VMEM((8, n), jnp.int32),
            pltpu.SMEM((8, n), jnp.int32),
            pltpu.SemaphoreType.DMA((_NBUF,)),
            pltpu.SemaphoreType.DMA,
        ],
        out_shape=jax.ShapeDtypeStruct((1, 1), jnp.float32),
    )(labr, labc, q, k, queue)
    return out[0, 0]
